# Initial kernel scaffold; baseline (speedup 1.0000x reference)
import jax
import jax.numpy as jnp
from jax.experimental import pallas as pl


def kernel(x, edge_index, edge_attr, u, batch, params):
    raise NotImplementedError("write your pallas kernel here")


# speedup vs baseline: 1.8564x; 1.8564x over previous
import functools

import jax
import jax.numpy as jnp
from jax import lax
from jax.experimental import pallas as pl
from jax.experimental.pallas import tpu as pltpu
from jax.experimental.pallas import tpu_sc as plsc

N_NODES = 10000
N_EDGES = 320000
D = 128
NP_NODES = 10240
NW = 32
EPT = N_EDGES // NW
EBK = 200
EBG = 80
NPT = NP_NODES // NW

_mesh = plsc.VectorSubcoreMesh(core_axis_name="c", subcore_axis_name="s")
_sc_params = pltpu.CompilerParams(needs_layout_passes=False)


def _wid():
    return lax.axis_index("s") * 2 + lax.axis_index("c")


def _lanes16(v, dtype=jnp.int32):
    return jnp.full((16,), v, dtype)


def _mm_body(a_ref, w_ref, b_ref, o_ref, *, relu):
    h = jnp.dot(a_ref[:], w_ref[:], preferred_element_type=jnp.float32)
    h = h + b_ref[:]
    if relu:
        h = jnp.maximum(h, 0.0)
    o_ref[:] = h


def _pick_bm(m):
    for bm in (2048, 2000, 1024, 1000, 512, 500, 256, 200, 128, 64, 16, 8):
        if m % bm == 0:
            return bm
    return m


def _mm(a, w, b, relu=False):
    m, k = a.shape
    n = w.shape[1]
    bm = _pick_bm(m)
    return pl.pallas_call(
        functools.partial(_mm_body, relu=relu),
        grid=(m // bm,),
        in_specs=[
            pl.BlockSpec((bm, k), lambda i: (i, 0)),
            pl.BlockSpec((k, n), lambda i: (0, 0)),
            pl.BlockSpec((1, n), lambda i: (0, 0)),
        ],
        out_specs=pl.BlockSpec((bm, n), lambda i: (i, 0)),
        out_shape=jax.ShapeDtypeStruct((m, n), jnp.float32),
    )(a, w, b.reshape(1, n))


@functools.partial(
    pl.kernel, mesh=_mesh, compiler_params=_sc_params,
    out_type=[jax.ShapeDtypeStruct((N_EDGES, D), jnp.float32),
              jax.ShapeDtypeStruct((N_EDGES, D), jnp.float32)],
    scratch_types=[pltpu.VMEM((EBG,), jnp.int32),
                   pltpu.VMEM((EBG, D), jnp.float32),
                   pltpu.SemaphoreType.DMA],
)
def _sc_gather(x_hbm, row_hbm, col_hbm, gr_hbm, gc_hbm, idx_v, rows_v, sem):
    base0 = _wid() * EPT

    def phase(idx_hbm, out_hbm):
        def body(j, carry):
            base = base0 + j * EBG
            pltpu.sync_copy(idx_hbm.at[pl.ds(base, EBG)], idx_v)
            pltpu.async_copy(x_hbm.at[idx_v], rows_v, sem).wait()
            pltpu.sync_copy(rows_v, out_hbm.at[pl.ds(base, EBG)])
            return carry
        lax.fori_loop(0, EPT // EBG, body, 0)

    phase(row_hbm, gr_hbm)
    phase(col_hbm, gc_hbm)


@functools.partial(
    pl.kernel, mesh=_mesh, compiler_params=_sc_params,
    out_type=[jax.ShapeDtypeStruct((NW, NP_NODES * 4), jnp.float32),
              jax.ShapeDtypeStruct((NW, NP_NODES * 3), jnp.float32)],
    scratch_types=[pltpu.VMEM((EBK,), jnp.float32),
                   pltpu.VMEM((EBK, 16), jnp.float32),
                   pltpu.VMEM((NP_NODES * 4,), jnp.float32),
                   pltpu.VMEM((NP_NODES * 3,), jnp.float32)],
)
def _sc_scatter_edges(vals_hbm, col_hbm, sum_hbm, max_hbm,
                      colf_v, vals_v, sum_f, max_f):
    w = _wid()
    base0 = w * EPT
    lane = lax.iota(jnp.int32, 16)
    zero16 = jnp.zeros((16,), jnp.float32)
    ninf16 = jnp.full((16,), -jnp.inf, jnp.float32)
    mask4 = lane < 4
    mask3 = lane < 3

    def init(i, carry):
        sum_f[pl.ds(i * 16, 16)] = zero16
        return carry
    lax.fori_loop(0, NP_NODES * 4 // 16, init, 0)

    def initm(i, carry):
        max_f[pl.ds(i * 16, 16)] = ninf16
        return carry
    lax.fori_loop(0, NP_NODES * 3 // 16, initm, 0)

    def chunk(j, carry):
        base = base0 + j * EBK
        pltpu.sync_copy(col_hbm.at[pl.ds(base, EBK)], colf_v)
        pltpu.sync_copy(vals_hbm.at[pl.ds(base, EBK)], vals_v)

        def edge(e, c2):
            colsp = plsc.load_gather(colf_v, [_lanes16(e)]).astype(jnp.int32)
            val = vals_v[e]
            idx4 = colsp * 4 + lane
            s = plsc.load_gather(sum_f, [idx4], mask=mask4)
            plsc.store_scatter(sum_f, [idx4], s + val, mask=mask4)
            idx3 = colsp * 3 + lane
            m = plsc.load_gather(max_f, [idx3], mask=mask3)
            plsc.store_scatter(max_f, [idx3], jnp.maximum(m, val), mask=mask3)
            return c2
        lax.fori_loop(0, EBK, edge, 0)
        return carry
    lax.fori_loop(0, EPT // EBK, chunk, 0)

    pltpu.sync_copy(sum_f, sum_hbm.at[w])
    pltpu.sync_copy(max_f, max_hbm.at[w])


@functools.partial(
    pl.kernel, mesh=_mesh, compiler_params=_sc_params,
    out_type=[jax.ShapeDtypeStruct((NW, 65 * D), jnp.float32),
              jax.ShapeDtypeStruct((NW, 65 * D), jnp.float32),
              jax.ShapeDtypeStruct((NW, 65 * 16), jnp.float32)],
    scratch_types=[pltpu.VMEM((NPT,), jnp.float32),
                   pltpu.VMEM((32, D), jnp.float32),
                   pltpu.VMEM((65 * D,), jnp.float32),
                   pltpu.VMEM((65 * D,), jnp.float32),
                   pltpu.VMEM((65 * 16,), jnp.float32)],
)
def _sc_pool(x_hbm, batch_hbm, sum_hbm, max_hbm, cnt_hbm,
             b_v, x_v, sum_f, max_f, cnt_f):
    w = _wid()
    base = w * NPT
    lane = lax.iota(jnp.int32, 16)
    zero16 = jnp.zeros((16,), jnp.float32)
    one16 = jnp.ones((16,), jnp.float32)
    ninf16 = jnp.full((16,), -jnp.inf, jnp.float32)

    def init(i, carry):
        sum_f[pl.ds(i * 16, 16)] = zero16
        max_f[pl.ds(i * 16, 16)] = ninf16
        return carry
    lax.fori_loop(0, 65 * D // 16, init, 0)

    def initc(i, carry):
        cnt_f[pl.ds(i * 16, 16)] = zero16
        return carry
    lax.fori_loop(0, 65, initc, 0)

    pltpu.sync_copy(batch_hbm.at[pl.ds(base, NPT)], b_v)

    def chunkp(j, carry):
        pltpu.sync_copy(x_hbm.at[pl.ds(base + j * 32, 32)], x_v)

        def node(i, c3):
            n = j * 32 + i
            bsp = plsc.load_gather(b_v, [_lanes16(n)]).astype(jnp.int32)
            idxc = bsp * 16 + lane
            c = plsc.load_gather(cnt_f, [idxc])
            plsc.store_scatter(cnt_f, [idxc], c + one16)
            for v in range(D // 16):
                val = x_v[i, pl.ds(v * 16, 16)]
                idx = bsp * D + v * 16 + lane
                s = plsc.load_gather(sum_f, [idx])
                plsc.store_scatter(sum_f, [idx], s + val)
                m = plsc.load_gather(max_f, [idx])
                plsc.store_scatter(max_f, [idx], jnp.maximum(m, val))
            return c3
        lax.fori_loop(0, 32, node, 0)
        return carry
    lax.fori_loop(0, NPT // 32, chunkp, 0)

    pltpu.sync_copy(sum_f, sum_hbm.at[w])
    pltpu.sync_copy(max_f, max_hbm.at[w])
    pltpu.sync_copy(cnt_f, cnt_hbm.at[w])


def kernel(x, edge_index, edge_attr, u, batch, params):
    row, col = edge_index[0], edge_index[1]
    n = x.shape[0]
    bsz = u.shape[0]

    for lp in params["layers"]:
        (w1, b1), (w2, b2) = lp["edge"]
        gr, gc = _sc_gather(x, row, col)
        e_in = jnp.concatenate([gr, gc, edge_attr], axis=1)
        h = _mm(e_in, w1, b1, relu=True)
        w2p = jnp.zeros((w2.shape[0], 16), jnp.float32).at[:, :3].set(w2)
        b2p = jnp.zeros((16,), jnp.float32).at[:3].set(b2).at[3].set(1.0)
        e3p = _mm(h, w2p, b2p)
        edge_attr = e3p[:, :3]
        sum_parts, max_parts = _sc_scatter_edges(e3p, col.astype(jnp.float32))
        sums = jnp.sum(sum_parts, axis=0).reshape(NP_NODES, 4)[:n]
        maxs = jnp.max(max_parts, axis=0).reshape(NP_NODES, 3)[:n]
        out1 = sums[:, :3]
        cnt = sums[:, 3:4]
        out2 = jnp.where(cnt > 0, maxs, 0.0)
        out3 = out1 / jnp.maximum(cnt, 1.0)
        (nw1, nb1), (nw2, nb2) = lp["node"]
        n_in = jnp.concatenate([x, out1, out2, out3, u[batch]], axis=1)
        x = _mm(_mm(n_in, nw1, nb1, relu=True), nw2, nb2)

    xpad = jnp.zeros((NP_NODES, D), jnp.float32).at[:n].set(x)
    bpad = jnp.full((NP_NODES,), bsz, jnp.float32).at[:n].set(batch.astype(jnp.float32))
    psum, pmax, pcnt = _sc_pool(xpad, bpad)
    addpool = jnp.sum(psum, axis=0).reshape(65, D)[:bsz]
    maxpool = jnp.max(pmax, axis=0).reshape(65, D)[:bsz]
    cntb = jnp.sum(pcnt, axis=0).reshape(65, 16)[:bsz, :1]
    meanpool = addpool / jnp.maximum(cntb, 1.0)
    maxpool = jnp.where(cntb > 0, maxpool, 0.0)
    out = jnp.concatenate([addpool, meanpool, maxpool, u], axis=1)
    for i, (w, b) in enumerate(params["out"]):
        out = _mm(out, w, b, relu=(i < 3))
    return out
